# baseline (speedup 1.0000x reference)
import functools

import jax
import jax.numpy as jnp
from jax import lax
from jax.experimental import pallas as pl
from jax.experimental.pallas import tpu as pltpu
from jax.experimental.pallas import tpu_sc as plsc

B = 16384
F = 26
NC = 2
NS = 16
NW = NC * NS
BW = B // NW
G = 512
TPW = BW // G
IPW = F * BW
LANES = 16


_mesh = plsc.VectorSubcoreMesh(core_axis_name="c", subcore_axis_name="s")


@functools.partial(
    pl.kernel,
    out_type=jax.ShapeDtypeStruct((B,), jnp.float32),
    mesh=_mesh,
    scratch_types=[
        pltpu.VMEM((F, TPW, G), jnp.int32),
        pltpu.VMEM((IPW,), jnp.float32),
        pltpu.VMEM((BW,), jnp.float32),
        pltpu.VMEM((LANES,), jnp.float32),
        pltpu.SemaphoreType.DMA,
    ],
)
def _lr_sc(xt3_hbm, tbl_hbm, bias_hbm, out_hbm, idx_v, rows_v, out_v,
           bias_v, sem):
    wid = lax.axis_index("c") * NS + lax.axis_index("s")
    base = wid * BW

    pltpu.sync_copy(bias_hbm, bias_v)
    pltpu.sync_copy(xt3_hbm.at[:, pl.ds(wid * TPW, TPW), :], idx_v)

    @pl.loop(0, F)
    def _fire(f):
        for t in range(TPW):
            pltpu.async_copy(
                tbl_hbm.at[idx_v.at[f, t]],
                rows_v.at[pl.ds(f * BW + t * G, G)],
                sem,
            )

    pltpu.make_async_copy(tbl_hbm.at[pl.ds(0, IPW)], rows_v, sem).wait()

    bvec = bias_v[pl.ds(0, LANES)]

    @pl.loop(0, BW // LANES)
    def _acc(c):
        off = c * LANES
        s = rows_v[pl.ds(off, LANES)]
        for f in range(1, F):
            s = s + rows_v[pl.ds(f * BW + off, LANES)]
        z = s + bvec
        out_v[pl.ds(off, LANES)] = 1.0 / (1.0 + jnp.exp(-z))

    pltpu.sync_copy(out_v, out_hbm.at[pl.ds(base, BW)])


def kernel(x, table, bias):
    xt3 = x.astype(jnp.int32).T.reshape(F, B // G, G)
    bias16 = jnp.broadcast_to(bias.astype(jnp.float32), (LANES,))
    out = _lr_sc(xt3, table.reshape(-1), bias16)
    return out.reshape(B, 1)


# speedup vs baseline: 1.0123x; 1.0123x over previous
import functools

import jax
import jax.numpy as jnp
from jax import lax
from jax.experimental import pallas as pl
from jax.experimental.pallas import tpu as pltpu
from jax.experimental.pallas import tpu_sc as plsc

B = 16384
F = 26
NC = 2
NS = 16
NW = NC * NS
BW = B // NW
G = 128
TPW = BW // G
IPW = F * BW
LANES = 16


_mesh = plsc.VectorSubcoreMesh(core_axis_name="c", subcore_axis_name="s")


@functools.partial(
    pl.kernel,
    out_type=jax.ShapeDtypeStruct((B,), jnp.float32),
    mesh=_mesh,
    scratch_types=[
        pltpu.VMEM((F, TPW, G), jnp.int32),
        pltpu.VMEM((IPW,), jnp.float32),
        pltpu.VMEM((BW,), jnp.float32),
        pltpu.VMEM((LANES,), jnp.float32),
        pltpu.SemaphoreType.DMA,
    ],
)
def _lr_sc(xt3_hbm, tbl_hbm, bias_hbm, out_hbm, idx_v, rows_v, out_v,
           bias_v, sem):
    wid = lax.axis_index("c") * NS + lax.axis_index("s")
    base = wid * BW

    pltpu.sync_copy(bias_hbm, bias_v)
    pltpu.sync_copy(xt3_hbm.at[:, pl.ds(wid * TPW, TPW), :], idx_v)

    @pl.loop(0, F)
    def _fire(f):
        for t in range(TPW):
            pltpu.async_copy(
                tbl_hbm.at[idx_v.at[f, t]],
                rows_v.at[pl.ds(f * BW + t * G, G)],
                sem,
            )

    pltpu.make_async_copy(tbl_hbm.at[pl.ds(0, IPW)], rows_v, sem).wait()

    bvec = bias_v[pl.ds(0, LANES)]

    @pl.loop(0, BW // LANES)
    def _acc(c):
        off = c * LANES
        s = rows_v[pl.ds(off, LANES)]
        for f in range(1, F):
            s = s + rows_v[pl.ds(f * BW + off, LANES)]
        z = s + bvec
        out_v[pl.ds(off, LANES)] = 1.0 / (1.0 + jnp.exp(-z))

    pltpu.sync_copy(out_v, out_hbm.at[pl.ds(base, BW)])


def kernel(x, table, bias):
    xt3 = x.astype(jnp.int32).T.reshape(F, B // G, G)
    bias16 = jnp.broadcast_to(bias.astype(jnp.float32), (LANES,))
    out = _lr_sc(xt3, table.reshape(-1), bias16)
    return out.reshape(B, 1)


# speedup vs baseline: 1.0297x; 1.0172x over previous
import functools

import jax
import jax.numpy as jnp
from jax import lax
from jax.experimental import pallas as pl
from jax.experimental.pallas import tpu as pltpu
from jax.experimental.pallas import tpu_sc as plsc

B = 16384
F = 26
NC = 2
NS = 16
NW = NC * NS
BW = B // NW
G = 128
TPW = BW // G
IPW = F * BW
LANES = 16


_mesh = plsc.VectorSubcoreMesh(core_axis_name="c", subcore_axis_name="s")


@functools.partial(
    pl.kernel,
    out_type=jax.ShapeDtypeStruct((B,), jnp.float32),
    mesh=_mesh,
    scratch_types=[
        pltpu.VMEM((F, TPW, G), jnp.int32),
        pltpu.VMEM((IPW,), jnp.float32),
        pltpu.VMEM((BW,), jnp.float32),
        pltpu.VMEM((BW,), jnp.float32),
        pltpu.VMEM((LANES,), jnp.float32),
        pltpu.SemaphoreType.DMA,
        pltpu.SemaphoreType.DMA((F,)),
    ],
)
def _lr_sc(xt3_hbm, tbl_hbm, bias_hbm, out_hbm, idx_v, rows_v, acc_v,
           out_v, bias_v, sem_i, sem_g):
    wid = lax.axis_index("c") * NS + lax.axis_index("s")
    base = wid * BW

    ci = pltpu.async_copy(xt3_hbm.at[:, pl.ds(wid * TPW, TPW), :], idx_v,
                          sem_i)
    cb = pltpu.async_copy(bias_hbm, bias_v, sem_i)
    ci.wait()
    cb.wait()

    @pl.loop(0, F)
    def _fire(f):
        for t in range(TPW):
            pltpu.async_copy(
                tbl_hbm.at[idx_v.at[f, t]],
                rows_v.at[pl.ds(f * BW + t * G, G)],
                sem_g.at[f],
            )

    def _wait_field(f):
        pltpu.make_async_copy(
            tbl_hbm.at[pl.ds(0, BW)], rows_v.at[pl.ds(f * BW, BW)],
            sem_g.at[f],
        ).wait()

    _wait_field(0)

    @pl.loop(0, BW // LANES)
    def _init(c):
        off = c * LANES
        acc_v[pl.ds(off, LANES)] = rows_v[pl.ds(off, LANES)]

    @pl.loop(1, F - 1)
    def _acc(f):
        _wait_field(f)

        @pl.loop(0, BW // LANES)
        def _add(c):
            off = c * LANES
            acc_v[pl.ds(off, LANES)] += rows_v[pl.ds(f * BW + off, LANES)]

    _wait_field(F - 1)
    bvec = bias_v[pl.ds(0, LANES)]

    @pl.loop(0, BW // LANES)
    def _fin(c):
        off = c * LANES
        z = (acc_v[pl.ds(off, LANES)]
             + rows_v[pl.ds((F - 1) * BW + off, LANES)] + bvec)
        out_v[pl.ds(off, LANES)] = 1.0 / (1.0 + jnp.exp(-z))

    pltpu.sync_copy(out_v, out_hbm.at[pl.ds(base, BW)])


def kernel(x, table, bias):
    xt3 = x.astype(jnp.int32).T.reshape(F, B // G, G)
    bias16 = jnp.broadcast_to(bias.astype(jnp.float32), (LANES,))
    out = _lr_sc(xt3, table.reshape(-1), bias16)
    return out.reshape(B, 1)


# speedup vs baseline: 1.0493x; 1.0190x over previous
import functools

import jax
import jax.numpy as jnp
from jax import lax
from jax.experimental import pallas as pl
from jax.experimental.pallas import tpu as pltpu
from jax.experimental.pallas import tpu_sc as plsc

B = 16384
F = 26
NC = 2
NS = 16
NW = NC * NS
BW = B // NW
G = 128
TPW = BW // G
IPW = F * BW
LANES = 16


_mesh = plsc.VectorSubcoreMesh(core_axis_name="c", subcore_axis_name="s")


@functools.partial(
    pl.kernel,
    out_type=jax.ShapeDtypeStruct((B,), jnp.float32),
    mesh=_mesh,
    scratch_types=[
        pltpu.VMEM((F, TPW, G), jnp.int32),
        pltpu.VMEM((IPW,), jnp.float32),
        pltpu.VMEM((BW,), jnp.float32),
        pltpu.VMEM((BW,), jnp.float32),
        pltpu.VMEM((LANES,), jnp.float32),
        pltpu.SemaphoreType.DMA,
        pltpu.SemaphoreType.DMA((F,)),
    ],
)
def _lr_sc(xt3_hbm, tbl_hbm, bias_hbm, out_hbm, idx_v, rows_v, acc_v,
           out_v, bias_v, sem_i, sem_g):
    wid = lax.axis_index("c") * NS + lax.axis_index("s")
    base = wid * BW

    ci = pltpu.async_copy(xt3_hbm.at[:, pl.ds(wid * TPW, TPW), :], idx_v,
                          sem_i)
    cb = pltpu.async_copy(bias_hbm, bias_v.at[pl.ds(0, 1)], sem_i)
    ci.wait()

    @pl.loop(0, F)
    def _fire(f):
        for t in range(TPW):
            pltpu.async_copy(
                tbl_hbm.at[idx_v.at[f, t]],
                rows_v.at[pl.ds(f * BW + t * G, G)],
                sem_g.at[f],
            )

    def _wait_field(f):
        pltpu.make_async_copy(
            tbl_hbm.at[pl.ds(0, BW)], rows_v.at[pl.ds(f * BW, BW)],
            sem_g.at[f],
        ).wait()

    _wait_field(0)

    @pl.loop(0, BW // LANES)
    def _init(c):
        off = c * LANES
        acc_v[pl.ds(off, LANES)] = rows_v[pl.ds(off, LANES)]

    @pl.loop(1, F - 1)
    def _acc(f):
        _wait_field(f)

        @pl.loop(0, BW // LANES)
        def _add(c):
            off = c * LANES
            acc_v[pl.ds(off, LANES)] += rows_v[pl.ds(f * BW + off, LANES)]

    _wait_field(F - 1)
    cb.wait()
    b = bias_v[pl.ds(0, LANES)][0]

    @pl.loop(0, BW // LANES)
    def _fin(c):
        off = c * LANES
        z = (acc_v[pl.ds(off, LANES)]
             + rows_v[pl.ds((F - 1) * BW + off, LANES)] + b)
        out_v[pl.ds(off, LANES)] = 1.0 / (1.0 + jnp.exp(-z))

    pltpu.sync_copy(out_v, out_hbm.at[pl.ds(base, BW)])


def kernel(x, table, bias):
    xt3 = x.astype(jnp.int32).T.reshape(F, B // G, G)
    out = _lr_sc(xt3, table.reshape(-1), bias.astype(jnp.float32))
    return out.reshape(B, 1)
